# baseline (speedup 1.0000x reference)
import jax
import jax.numpy as jnp
from jax import lax
from jax.experimental import pallas as pl
from jax.experimental.pallas import tpu as pltpu
from jax.experimental.pallas import tpu_sc as plsc

N_NODES = 10000
N_EDGES = 160000
HIDDEN_X = 256
HIDDEN_T = 128
HALF = 128

NC = 2
NS = 16
CHUNK = 32
BLK_CHUNKS = 8
N_BLOCKS = 40
EDGES_PER_TILE = CHUNK * BLK_CHUNKS * N_BLOCKS
E_PAD = EDGES_PER_TILE * NS
N_PASS = 2
PASS_ROWS = 5000
HX_ROWS = 5008
ZERO_LOCAL = PASS_ROWS
ACC_ROWS = 10008
DUMP_ROW = N_NODES
STEP = 624
HSTEP = 312


def _sc_body(hx0, hx1, srcp, dstp, zinit, aggr,
             src_v, dst_v, rows0, rows1, hxs, acc, sem0, sem1):
    c = lax.axis_index("c")
    s = lax.axis_index("s")

    pltpu.sync_copy(zinit, acc.at[pl.ds(s * STEP, STEP)])

    @pl.when(s == 0)
    def _():
        pltpu.sync_copy(zinit.at[pl.ds(0, 24)], acc.at[pl.ds(9984, 24)])

    def run(hx):
        def one_pass(p, _):
            lo = p * PASS_ROWS
            plsc.subcore_barrier()
            pltpu.sync_copy(
                hx.at[pl.ds(pl.multiple_of(lo + s * HSTEP, 8), HSTEP)],
                hxs.at[pl.ds(s * HSTEP, HSTEP)])

            @pl.when(s == 0)
            def _():
                pltpu.sync_copy(hx.at[pl.ds(pl.multiple_of(lo + 4992, 8), 8)],
                                hxs.at[pl.ds(4992, 8)])
                pltpu.sync_copy(zinit.at[pl.ds(0, 8)],
                                hxs.at[pl.ds(ZERO_LOCAL, 8)])

            plsc.subcore_barrier()

            def block(sb, _):
                pltpu.sync_copy(srcp.at[s, sb], src_v)
                pltpu.sync_copy(dstp.at[s, sb], dst_v)
                for r in range(BLK_CHUNKS):
                    for k in range(CHUNK // 16):
                        sl = (pl.ds(r, 1), pl.ds(k * 16, 16))
                        raw = src_v[sl].reshape(16)
                        t = raw - lo
                        ok = (t >= 0) & (t < PASS_ROWS)
                        src_v[sl] = jnp.where(ok, t, ZERO_LOCAL).reshape(
                            1, 16)

                pltpu.async_copy(hxs.at[src_v.at[0]], rows0, sem0)

                def step(i, _):
                    j = 2 * i
                    cp1 = pltpu.async_copy(hxs.at[src_v.at[j + 1]], rows1,
                                           sem1)
                    pltpu.make_async_copy(hxs.at[src_v.at[j]], rows0,
                                          sem0).wait()
                    pltpu.sync_copy(rows0, acc.at[dst_v.at[j]], add=True)

                    @pl.when(j + 2 < BLK_CHUNKS)
                    def _():
                        pltpu.async_copy(hxs.at[src_v.at[j + 2]], rows0,
                                         sem0)

                    cp1.wait()
                    pltpu.sync_copy(rows1, acc.at[dst_v.at[j + 1]],
                                    add=True)
                    return 0

                lax.fori_loop(0, BLK_CHUNKS // 2, step, 0)
                return 0

            lax.fori_loop(0, N_BLOCKS, block, 0)
            return 0

        lax.fori_loop(0, N_PASS, one_pass, 0)

    @pl.when(c == 0)
    def _():
        run(hx0)

    @pl.when(c == 1)
    def _():
        run(hx1)

    plsc.subcore_barrier()
    pltpu.sync_copy(acc.at[pl.ds(s * STEP, STEP)],
                    aggr.at[pl.ds(s * STEP, STEP), pl.ds(c * HALF, HALF)])

    @pl.when(s == 0)
    def _():
        pltpu.sync_copy(acc.at[pl.ds(9984, 16)],
                        aggr.at[pl.ds(9984, 16), pl.ds(c * HALF, HALF)])


def _sc_aggregate(hx0, hx1, srcp, dstp, zinit):
    mesh = plsc.VectorSubcoreMesh(core_axis_name="c", subcore_axis_name="s")
    return pl.kernel(
        _sc_body,
        out_type=jax.ShapeDtypeStruct((N_NODES, HIDDEN_X), jnp.float32),
        mesh=mesh,
        scratch_types=[
            pltpu.VMEM((BLK_CHUNKS, CHUNK), jnp.int32),
            pltpu.VMEM((BLK_CHUNKS, CHUNK), jnp.int32),
            pltpu.VMEM((CHUNK, HALF), jnp.float32),
            pltpu.VMEM((CHUNK, HALF), jnp.float32),
            pltpu.VMEM_SHARED((HX_ROWS, HALF), jnp.float32),
            pltpu.VMEM_SHARED((ACC_ROWS, HALF), jnp.float32),
            pltpu.SemaphoreType.DMA,
            pltpu.SemaphoreType.DMA,
        ],
    )(hx0, hx1, srcp, dstp, zinit)


def _tc_body(a_ref, ht_ref, wT_ref, b_ref, g_ref, bt_ref, o_ref):
    z = lax.dot_general(a_ref[:, :], wT_ref[:HIDDEN_X, :],
                        (((1,), (0,)), ((), ())),
                        preferred_element_type=jnp.float32)
    ct = lax.dot_general(ht_ref[:, :], wT_ref[HIDDEN_X:, :],
                         (((1,), (0,)), ((), ())),
                         preferred_element_type=jnp.float32)
    z = z + ct + b_ref[:, :]
    z = jnp.maximum(z, 0.0)
    mean = jnp.mean(z, axis=1, keepdims=True)
    zc = z - mean
    var = jnp.mean(zc * zc, axis=1, keepdims=True)
    z = zc * lax.rsqrt(var + 1e-5) * g_ref[:, :] + bt_ref[:, :]
    o_ref[:, :] = z


def _tc_update(aggr, h_t, wT, b, gamma, beta):
    blk = 1000
    grid = N_NODES // blk
    return pl.pallas_call(
        _tc_body,
        grid=(grid,),
        in_specs=[
            pl.BlockSpec((blk, HIDDEN_X), lambda i: (i, 0)),
            pl.BlockSpec((1, HIDDEN_T), lambda i: (0, 0)),
            pl.BlockSpec((HIDDEN_X + HIDDEN_T, HIDDEN_X), lambda i: (0, 0)),
            pl.BlockSpec((1, HIDDEN_X), lambda i: (0, 0)),
            pl.BlockSpec((1, HIDDEN_X), lambda i: (0, 0)),
            pl.BlockSpec((1, HIDDEN_X), lambda i: (0, 0)),
        ],
        out_specs=pl.BlockSpec((blk, HIDDEN_X), lambda i: (i, 0)),
        out_shape=jax.ShapeDtypeStruct((N_NODES, HIDDEN_X), jnp.float32),
    )(aggr, h_t, wT, b, gamma, beta)


@jax.jit
def kernel(edge_index, h_X, h_t, W, b, gamma, beta):
    src = edge_index[0]
    dst = edge_index[1]
    pad = E_PAD - N_EDGES
    srcp = jnp.concatenate([src, jnp.zeros((pad,), jnp.int32)])
    dstp = jnp.concatenate([dst, jnp.full((pad,), DUMP_ROW, jnp.int32)])
    srcp = srcp.reshape(NS, N_BLOCKS, BLK_CHUNKS, CHUNK)
    dstp = dstp.reshape(NS, N_BLOCKS, BLK_CHUNKS, CHUNK)
    hx0 = h_X[:, :HALF]
    hx1 = h_X[:, HALF:]
    zinit = jnp.zeros((STEP, HALF), jnp.float32)

    aggr = _sc_aggregate(hx0, hx1, srcp, dstp, zinit)

    wT = W.T
    return _tc_update(aggr, h_t, wT,
                      b.reshape(1, HIDDEN_X),
                      gamma.reshape(1, HIDDEN_X),
                      beta.reshape(1, HIDDEN_X))


# speedup vs baseline: 1.3774x; 1.3774x over previous
import functools

import jax
import jax.numpy as jnp
from jax import lax
from jax.experimental import pallas as pl
from jax.experimental.pallas import tpu as pltpu
from jax.experimental.pallas import tpu_sc as plsc

N_NODES = 10000
N_EDGES = 160000
HIDDEN_X = 256
HIDDEN_T = 128
HALF = 128

NC = 2
NS = 16
CHUNK = 128
BLK_CHUNKS = 16
N_STAGES = 5
EDGES_PER_TILE = CHUNK * BLK_CHUNKS * N_STAGES
N_CHUNKS = EDGES_PER_TILE // CHUNK
E_PAD = EDGES_PER_TILE * NS
ACC_ROWS = 10240
ROWS_PER_TILE_INIT = ACC_ROWS // NS


def _sc_body(hx0, hx1, srcp, dstp, zinit, aggr,
             src_v, dst_v, rows0, rows1, acc, sem0, sem1):
    c = lax.axis_index("c")
    s = lax.axis_index("s")

    pltpu.sync_copy(zinit, acc.at[pl.ds(s * ROWS_PER_TILE_INIT,
                                        ROWS_PER_TILE_INIT)])
    plsc.subcore_barrier()

    def run(hx):
        def stage(st, _):
            pltpu.sync_copy(srcp.at[s, st], src_v)
            pltpu.sync_copy(dstp.at[s, st], dst_v)
            pltpu.async_copy(hx.at[src_v.at[0]], rows0, sem0)

            def step(i, _):
                j = 2 * i
                cp1 = pltpu.async_copy(hx.at[src_v.at[j + 1]], rows1, sem1)
                pltpu.make_async_copy(hx.at[src_v.at[j]], rows0, sem0).wait()
                pltpu.sync_copy(rows0, acc.at[dst_v.at[j]], add=True)

                @pl.when(j + 2 < BLK_CHUNKS)
                def _():
                    pltpu.async_copy(hx.at[src_v.at[j + 2]], rows0, sem0)

                cp1.wait()
                pltpu.sync_copy(rows1, acc.at[dst_v.at[j + 1]], add=True)
                return 0

            lax.fori_loop(0, BLK_CHUNKS // 2, step, 0)
            return 0

        lax.fori_loop(0, N_STAGES, stage, 0)

    @pl.when(c == 0)
    def _():
        run(hx0.at[:, pl.ds(0, HALF)])

    @pl.when(c == 1)
    def _():
        run(hx0.at[:, pl.ds(HALF, HALF)])

    plsc.subcore_barrier()
    r0 = s * ROWS_PER_TILE_INIT
    pltpu.sync_copy(acc.at[pl.ds(r0, ROWS_PER_TILE_INIT)],
                    aggr.at[pl.ds(r0, ROWS_PER_TILE_INIT),
                            pl.ds(c * HALF, HALF)])


def _sc_aggregate(hx0, hx1, srcp, dstp, zinit):
    mesh = plsc.VectorSubcoreMesh(core_axis_name="c", subcore_axis_name="s")
    return pl.kernel(
        _sc_body,
        out_type=jax.ShapeDtypeStruct((ACC_ROWS, HIDDEN_X), jnp.float32),
        mesh=mesh,
        scratch_types=[
            pltpu.VMEM((BLK_CHUNKS, CHUNK), jnp.int32),
            pltpu.VMEM((BLK_CHUNKS, CHUNK), jnp.int32),
            pltpu.VMEM((CHUNK, HALF), jnp.float32),
            pltpu.VMEM((CHUNK, HALF), jnp.float32),
            pltpu.VMEM_SHARED((ACC_ROWS, HALF), jnp.float32),
            pltpu.SemaphoreType.DMA,
            pltpu.SemaphoreType.DMA,
        ],
    )(hx0, hx1, srcp, dstp, zinit)


def _tc_body(a_ref, ht_ref, wT_ref, b_ref, g_ref, bt_ref, o_ref):
    z = lax.dot_general(a_ref[:, :], wT_ref[:HIDDEN_X, :],
                        (((1,), (0,)), ((), ())),
                        preferred_element_type=jnp.float32)
    ct = lax.dot_general(ht_ref[:, :], wT_ref[HIDDEN_X:, :],
                         (((1,), (0,)), ((), ())),
                         preferred_element_type=jnp.float32)
    z = z + ct + b_ref[:, :]
    z = jnp.maximum(z, 0.0)
    mean = jnp.mean(z, axis=1, keepdims=True)
    zc = z - mean
    var = jnp.mean(zc * zc, axis=1, keepdims=True)
    z = zc * lax.rsqrt(var + 1e-5) * g_ref[:, :] + bt_ref[:, :]
    o_ref[:, :] = z


def _tc_update(aggr, h_t, wT, b, gamma, beta):
    blk = 1000
    grid = N_NODES // blk
    return pl.pallas_call(
        _tc_body,
        grid=(grid,),
        in_specs=[
            pl.BlockSpec((blk, HIDDEN_X), lambda i: (i, 0)),
            pl.BlockSpec((1, HIDDEN_T), lambda i: (0, 0)),
            pl.BlockSpec((HIDDEN_X + HIDDEN_T, HIDDEN_X), lambda i: (0, 0)),
            pl.BlockSpec((1, HIDDEN_X), lambda i: (0, 0)),
            pl.BlockSpec((1, HIDDEN_X), lambda i: (0, 0)),
            pl.BlockSpec((1, HIDDEN_X), lambda i: (0, 0)),
        ],
        out_specs=pl.BlockSpec((blk, HIDDEN_X), lambda i: (i, 0)),
        out_shape=jax.ShapeDtypeStruct((N_NODES, HIDDEN_X), jnp.float32),
    )(aggr, h_t, wT, b, gamma, beta)


@jax.jit
def kernel(edge_index, h_X, h_t, W, b, gamma, beta):
    src = edge_index[0]
    dst = edge_index[1]
    pad = E_PAD - N_EDGES
    srcp = jnp.concatenate([src, jnp.zeros((pad,), jnp.int32)])
    dstp = jnp.concatenate([dst,
                            jnp.full((pad,), ACC_ROWS - 1, jnp.int32)])
    srcp = srcp.reshape(NS, N_STAGES, BLK_CHUNKS, CHUNK)
    dstp = dstp.reshape(NS, N_STAGES, BLK_CHUNKS, CHUNK)
    hx0 = h_X
    hx1 = h_X
    zinit = jnp.zeros((ROWS_PER_TILE_INIT, HALF), jnp.float32)

    aggr = _sc_aggregate(hx0, hx1, srcp, dstp, zinit)

    wT = W.T
    return _tc_update(aggr, h_t, wT,
                      b.reshape(1, HIDDEN_X),
                      gamma.reshape(1, HIDDEN_X),
                      beta.reshape(1, HIDDEN_X))


# speedup vs baseline: 1.3879x; 1.0076x over previous
import functools

import jax
import jax.numpy as jnp
from jax import lax
from jax.experimental import pallas as pl
from jax.experimental.pallas import tpu as pltpu
from jax.experimental.pallas import tpu_sc as plsc

N_NODES = 10000
N_EDGES = 160000
HIDDEN_X = 256
HIDDEN_T = 128
HALF = 128

NC = 2
NS = 16
CHUNK = 128
BLK_CHUNKS = 16
N_STAGES = 5
EDGES_PER_TILE = CHUNK * BLK_CHUNKS * N_STAGES
N_CHUNKS = EDGES_PER_TILE // CHUNK
E_PAD = EDGES_PER_TILE * NS
ACC_ROWS = 10240
ROWS_PER_TILE_INIT = ACC_ROWS // NS


def _sc_body(hx0, hx1, srcp, dstp, zinit, aggr,
             src_v, dst_v, rows0, rows1, acc, sem0, sem1):
    c = lax.axis_index("c")
    s = lax.axis_index("s")

    pltpu.sync_copy(zinit, acc.at[pl.ds(s * ROWS_PER_TILE_INIT,
                                        ROWS_PER_TILE_INIT)])
    plsc.subcore_barrier()

    def run(hx):
        def stage(st, _):
            pltpu.sync_copy(srcp.at[s, st], src_v)
            pltpu.sync_copy(dstp.at[s, st], dst_v)
            pltpu.async_copy(hx.at[src_v.at[0]], rows0, sem0)

            def step(i, _):
                j = 2 * i
                cp1 = pltpu.async_copy(hx.at[src_v.at[j + 1]], rows1, sem1)
                pltpu.make_async_copy(hx.at[src_v.at[j]], rows0, sem0).wait()
                pltpu.sync_copy(rows0, acc.at[dst_v.at[j]], add=True)

                @pl.when(j + 2 < BLK_CHUNKS)
                def _():
                    pltpu.async_copy(hx.at[src_v.at[j + 2]], rows0, sem0)

                cp1.wait()
                pltpu.sync_copy(rows1, acc.at[dst_v.at[j + 1]], add=True)
                return 0

            lax.fori_loop(0, BLK_CHUNKS // 2, step, 0)
            return 0

        lax.fori_loop(0, N_STAGES, stage, 0)

    @pl.when(c == 0)
    def _():
        run(hx0.at[:, pl.ds(0, HALF)])

    @pl.when(c == 1)
    def _():
        run(hx0.at[:, pl.ds(HALF, HALF)])

    plsc.subcore_barrier()
    r0 = s * ROWS_PER_TILE_INIT
    pltpu.sync_copy(acc.at[pl.ds(r0, ROWS_PER_TILE_INIT)],
                    aggr.at[pl.ds(r0, ROWS_PER_TILE_INIT),
                            pl.ds(c * HALF, HALF)])


def _sc_aggregate(hx0, hx1, srcp, dstp, zinit):
    mesh = plsc.VectorSubcoreMesh(core_axis_name="c", subcore_axis_name="s")
    return pl.kernel(
        _sc_body,
        out_type=jax.ShapeDtypeStruct((ACC_ROWS, HIDDEN_X), jnp.float32),
        mesh=mesh,
        scratch_types=[
            pltpu.VMEM((BLK_CHUNKS, CHUNK), jnp.int32),
            pltpu.VMEM((BLK_CHUNKS, CHUNK), jnp.int32),
            pltpu.VMEM((CHUNK, HALF), jnp.float32),
            pltpu.VMEM((CHUNK, HALF), jnp.float32),
            pltpu.VMEM_SHARED((ACC_ROWS, HALF), jnp.float32),
            pltpu.SemaphoreType.DMA,
            pltpu.SemaphoreType.DMA,
        ],
    )(hx0, hx1, srcp, dstp, zinit)


def _tc_body(a_ref, ht_ref, wT_ref, b_ref, g_ref, bt_ref, o_ref):
    z = lax.dot_general(a_ref[:, :], wT_ref[:HIDDEN_X, :],
                        (((1,), (0,)), ((), ())),
                        preferred_element_type=jnp.float32)
    ct = lax.dot_general(ht_ref[:, :], wT_ref[HIDDEN_X:, :],
                         (((1,), (0,)), ((), ())),
                         preferred_element_type=jnp.float32)
    z = z + ct + b_ref[:, :]
    z = jnp.maximum(z, 0.0)
    mean = jnp.mean(z, axis=1, keepdims=True)
    zc = z - mean
    var = jnp.mean(zc * zc, axis=1, keepdims=True)
    z = zc * lax.rsqrt(var + 1e-5) * g_ref[:, :] + bt_ref[:, :]
    o_ref[:, :] = z


def _tc_update(aggr, h_t, wT, b, gamma, beta):
    blk = 2000
    grid = N_NODES // blk
    return pl.pallas_call(
        _tc_body,
        grid=(grid,),
        in_specs=[
            pl.BlockSpec((blk, HIDDEN_X), lambda i: (i, 0)),
            pl.BlockSpec((1, HIDDEN_T), lambda i: (0, 0)),
            pl.BlockSpec((HIDDEN_X + HIDDEN_T, HIDDEN_X), lambda i: (0, 0)),
            pl.BlockSpec((1, HIDDEN_X), lambda i: (0, 0)),
            pl.BlockSpec((1, HIDDEN_X), lambda i: (0, 0)),
            pl.BlockSpec((1, HIDDEN_X), lambda i: (0, 0)),
        ],
        out_specs=pl.BlockSpec((blk, HIDDEN_X), lambda i: (i, 0)),
        out_shape=jax.ShapeDtypeStruct((N_NODES, HIDDEN_X), jnp.float32),
    )(aggr, h_t, wT, b, gamma, beta)


@jax.jit
def kernel(edge_index, h_X, h_t, W, b, gamma, beta):
    src = edge_index[0]
    dst = edge_index[1]
    pad = E_PAD - N_EDGES
    srcp = jnp.concatenate([src, jnp.zeros((pad,), jnp.int32)])
    dstp = jnp.concatenate([dst,
                            jnp.full((pad,), ACC_ROWS - 1, jnp.int32)])
    srcp = srcp.reshape(NS, N_STAGES, BLK_CHUNKS, CHUNK)
    dstp = dstp.reshape(NS, N_STAGES, BLK_CHUNKS, CHUNK)
    hx0 = h_X
    hx1 = h_X
    zinit = jnp.zeros((ROWS_PER_TILE_INIT, HALF), jnp.float32)

    aggr = _sc_aggregate(hx0, hx1, srcp, dstp, zinit)

    wT = W.T
    return _tc_update(aggr, h_t, wT,
                      b.reshape(1, HIDDEN_X),
                      gamma.reshape(1, HIDDEN_X),
                      beta.reshape(1, HIDDEN_X))


# speedup vs baseline: 1.4130x; 1.0181x over previous
import functools

import jax
import jax.numpy as jnp
from jax import lax
from jax.experimental import pallas as pl
from jax.experimental.pallas import tpu as pltpu
from jax.experimental.pallas import tpu_sc as plsc

N_NODES = 10000
N_EDGES = 160000
HIDDEN_X = 256
HIDDEN_T = 128
HALF = 128

NC = 2
NS = 16
CHUNK = 128
BLK_CHUNKS = 40
N_STAGES = 2
EDGES_PER_TILE = CHUNK * BLK_CHUNKS * N_STAGES
N_CHUNKS = EDGES_PER_TILE // CHUNK
E_PAD = EDGES_PER_TILE * NS
ACC_ROWS = 10240
ROWS_PER_TILE_INIT = ACC_ROWS // NS


def _sc_body(hx0, hx1, srcp, dstp, zinit, aggr,
             src_v, dst_v, rows0, rows1, acc, sem0, sem1):
    c = lax.axis_index("c")
    s = lax.axis_index("s")

    pltpu.sync_copy(zinit, acc.at[pl.ds(s * ROWS_PER_TILE_INIT,
                                        ROWS_PER_TILE_INIT)])
    plsc.subcore_barrier()

    def run(hx):
        def stage(st, _):
            pltpu.sync_copy(srcp.at[s, st], src_v)
            pltpu.sync_copy(dstp.at[s, st], dst_v)
            pltpu.async_copy(hx.at[src_v.at[0]], rows0, sem0)

            def step(i, _):
                j = 2 * i
                cp1 = pltpu.async_copy(hx.at[src_v.at[j + 1]], rows1, sem1)
                pltpu.make_async_copy(hx.at[src_v.at[j]], rows0, sem0).wait()
                pltpu.sync_copy(rows0, acc.at[dst_v.at[j]], add=True)

                @pl.when(j + 2 < BLK_CHUNKS)
                def _():
                    pltpu.async_copy(hx.at[src_v.at[j + 2]], rows0, sem0)

                cp1.wait()
                pltpu.sync_copy(rows1, acc.at[dst_v.at[j + 1]], add=True)
                return 0

            lax.fori_loop(0, BLK_CHUNKS // 2, step, 0)
            return 0

        lax.fori_loop(0, N_STAGES, stage, 0)

    @pl.when(c == 0)
    def _():
        run(hx0.at[:, pl.ds(0, HALF)])

    @pl.when(c == 1)
    def _():
        run(hx0.at[:, pl.ds(HALF, HALF)])

    plsc.subcore_barrier()
    r0 = s * ROWS_PER_TILE_INIT
    pltpu.sync_copy(acc.at[pl.ds(r0, ROWS_PER_TILE_INIT)],
                    aggr.at[pl.ds(r0, ROWS_PER_TILE_INIT),
                            pl.ds(c * HALF, HALF)])


def _sc_aggregate(hx0, hx1, srcp, dstp, zinit):
    mesh = plsc.VectorSubcoreMesh(core_axis_name="c", subcore_axis_name="s")
    return pl.kernel(
        _sc_body,
        out_type=jax.ShapeDtypeStruct((ACC_ROWS, HIDDEN_X), jnp.float32),
        mesh=mesh,
        scratch_types=[
            pltpu.VMEM((BLK_CHUNKS, CHUNK), jnp.int32),
            pltpu.VMEM((BLK_CHUNKS, CHUNK), jnp.int32),
            pltpu.VMEM((CHUNK, HALF), jnp.float32),
            pltpu.VMEM((CHUNK, HALF), jnp.float32),
            pltpu.VMEM_SHARED((ACC_ROWS, HALF), jnp.float32),
            pltpu.SemaphoreType.DMA,
            pltpu.SemaphoreType.DMA,
        ],
    )(hx0, hx1, srcp, dstp, zinit)


def _tc_body(a_ref, ht_ref, wT_ref, b_ref, g_ref, bt_ref, o_ref):
    z = lax.dot_general(a_ref[:, :], wT_ref[:HIDDEN_X, :],
                        (((1,), (0,)), ((), ())),
                        preferred_element_type=jnp.float32)
    ct = lax.dot_general(ht_ref[:, :], wT_ref[HIDDEN_X:, :],
                         (((1,), (0,)), ((), ())),
                         preferred_element_type=jnp.float32)
    z = z + ct + b_ref[:, :]
    z = jnp.maximum(z, 0.0)
    mean = jnp.mean(z, axis=1, keepdims=True)
    zc = z - mean
    var = jnp.mean(zc * zc, axis=1, keepdims=True)
    z = zc * lax.rsqrt(var + 1e-5) * g_ref[:, :] + bt_ref[:, :]
    o_ref[:, :] = z


def _tc_update(aggr, h_t, wT, b, gamma, beta):
    blk = 2000
    grid = N_NODES // blk
    return pl.pallas_call(
        _tc_body,
        grid=(grid,),
        in_specs=[
            pl.BlockSpec((blk, HIDDEN_X), lambda i: (i, 0)),
            pl.BlockSpec((1, HIDDEN_T), lambda i: (0, 0)),
            pl.BlockSpec((HIDDEN_X + HIDDEN_T, HIDDEN_X), lambda i: (0, 0)),
            pl.BlockSpec((1, HIDDEN_X), lambda i: (0, 0)),
            pl.BlockSpec((1, HIDDEN_X), lambda i: (0, 0)),
            pl.BlockSpec((1, HIDDEN_X), lambda i: (0, 0)),
        ],
        out_specs=pl.BlockSpec((blk, HIDDEN_X), lambda i: (i, 0)),
        out_shape=jax.ShapeDtypeStruct((N_NODES, HIDDEN_X), jnp.float32),
    )(aggr, h_t, wT, b, gamma, beta)


@jax.jit
def kernel(edge_index, h_X, h_t, W, b, gamma, beta):
    src = edge_index[0]
    dst = edge_index[1]
    pad = E_PAD - N_EDGES
    srcp = jnp.concatenate([src, jnp.zeros((pad,), jnp.int32)])
    dstp = jnp.concatenate([dst,
                            jnp.full((pad,), ACC_ROWS - 1, jnp.int32)])
    srcp = srcp.reshape(NS, N_STAGES, BLK_CHUNKS, CHUNK)
    dstp = dstp.reshape(NS, N_STAGES, BLK_CHUNKS, CHUNK)
    hx0 = h_X
    hx1 = h_X
    zinit = jnp.zeros((ROWS_PER_TILE_INIT, HALF), jnp.float32)

    aggr = _sc_aggregate(hx0, hx1, srcp, dstp, zinit)

    wT = W.T
    return _tc_update(aggr, h_t, wT,
                      b.reshape(1, HIDDEN_X),
                      gamma.reshape(1, HIDDEN_X),
                      beta.reshape(1, HIDDEN_X))
